# baseline (speedup 1.0000x reference)
import functools

import jax
import jax.numpy as jnp
from jax.experimental import pallas as pl
from jax.experimental.pallas import tpu as pltpu

SMOOTH = 0.1
PAD = 0
CONF = 1.0 - SMOOTH

BR = 256
BC = 8192
LN = 128


def _loss_kernel(x_ref, y_ref, out_ref, m_s, s_s, rs_s, xy_s, x0_s, *, c, ncb):
    j = pl.program_id(1)

    @pl.when(j == 0)
    def _init():
        s_s[...] = jnp.zeros_like(s_s)
        rs_s[...] = jnp.zeros_like(rs_s)
        xy_s[...] = jnp.zeros_like(xy_s)
        m_s[...] = jnp.full_like(m_s, -jnp.inf)
        x0_s[...] = x_ref[:, 0:1]

    yv = y_ref[...]
    lane = jax.lax.broadcasted_iota(jnp.int32, (BR, LN), 1)

    def block(nch, tail):
        bm = x_ref[:, 0:LN]
        for k in range(1, nch):
            bm = jnp.maximum(bm, x_ref[:, k * LN:(k + 1) * LN])
        if tail:
            tl = jnp.where(lane < tail,
                           x_ref[:, nch * LN:(nch + 1) * LN], -jnp.inf)
            bm = jnp.maximum(bm, tl)
        new_m = jnp.maximum(m_s[...], bm)
        s_s[...] = s_s[...] * jnp.exp(m_s[...] - new_m)
        m_s[...] = new_m

        se = s_s[...]
        rs = rs_s[...]
        xy = xy_s[...]
        ybb = jnp.broadcast_to(yv - j * BC, (BR, LN))
        for k in range(nch + (1 if tail else 0)):
            raw = x_ref[:, k * LN:(k + 1) * LN]
            if k == nch:
                ch_e = jnp.where(lane < tail, raw, -jnp.inf)
                ch_z = jnp.where(lane < tail, raw, 0.0)
            else:
                ch_e = ch_z = raw
            se = se + jnp.exp(ch_e - new_m)
            rs = rs + ch_z
            xy = xy + jnp.where(ybb - k * LN == lane, ch_z, 0.0)
        s_s[...] = se
        rs_s[...] = rs
        xy_s[...] = xy

    @pl.when(j < ncb - 1)
    def _full():
        block(BC // LN, 0)

    @pl.when(j == ncb - 1)
    def _partial():
        rem = c - (ncb - 1) * BC
        block(rem // LN, rem % LN)

        eps = SMOOTH / (c - 2)
        k_const = SMOOTH * jnp.log(jnp.float32(eps)) + CONF * jnp.log(
            jnp.float32(CONF))
        mm = m_s[...]
        big_m = jnp.max(mm, axis=1, keepdims=True)
        s = jnp.sum(s_s[...] * jnp.exp(mm - big_m), axis=1, keepdims=True)
        lse = big_m + jnp.log(s)
        xyv = jnp.sum(xy_s[...], axis=1, keepdims=True)
        rowsum = jnp.sum(rs_s[...], axis=1, keepdims=True)
        rest = rowsum - x0_s[...] - xyv - (c - 2) * lse
        row = k_const - eps * rest - CONF * (xyv - lse)
        row = jnp.where(yv != PAD, row, 0.0)
        out_ref[...] = jnp.sum(row, keepdims=True)[None]


@jax.jit
def kernel(x, y):
    b, c = x.shape
    ncb = pl.cdiv(c, BC)
    nrb = b // BR
    y2 = y.astype(jnp.int32).reshape(b, 1)
    parts = pl.pallas_call(
        functools.partial(_loss_kernel, c=c, ncb=ncb),
        grid=(nrb, ncb),
        in_specs=[
            pl.BlockSpec((BR, BC), lambda i, j: (i, j)),
            pl.BlockSpec((BR, 1), lambda i, j: (i, 0)),
        ],
        out_specs=pl.BlockSpec((1, 1, 1), lambda i, j: (i, 0, 0)),
        out_shape=jax.ShapeDtypeStruct((nrb, 1, 1), jnp.float32),
        scratch_shapes=[
            pltpu.VMEM((BR, LN), jnp.float32),
            pltpu.VMEM((BR, LN), jnp.float32),
            pltpu.VMEM((BR, LN), jnp.float32),
            pltpu.VMEM((BR, LN), jnp.float32),
            pltpu.VMEM((BR, 1), jnp.float32),
        ],
        compiler_params=pltpu.CompilerParams(
            dimension_semantics=("parallel", "arbitrary")),
    )(x, y2)
    return jnp.sum(parts)


# speedup vs baseline: 1.1568x; 1.1568x over previous
import functools

import jax
import jax.numpy as jnp
from jax.experimental import pallas as pl
from jax.experimental.pallas import tpu as pltpu

SMOOTH = 0.1
PAD = 0
CONF = 1.0 - SMOOTH

BR = 256
BC = 8192
LN = 128


def _loss_kernel(x_ref, y_ref, out_ref, m_s, s_s, rs_s, xy_s, x0_s, *, c, ncb):
    j = pl.program_id(1)

    @pl.when(j == 0)
    def _init():
        s_s[...] = jnp.zeros_like(s_s)
        rs_s[...] = jnp.zeros_like(rs_s)
        xy_s[...] = jnp.zeros_like(xy_s)
        m_s[...] = jnp.full_like(m_s, -jnp.inf)
        x0_s[...] = x_ref[:, 0:1]

    yv = y_ref[...]
    lane = jax.lax.broadcasted_iota(jnp.int32, (BR, LN), 1)

    def block(nch, tail):
        bm = x_ref[:, 0:LN]
        for k in range(1, nch):
            bm = jnp.maximum(bm, x_ref[:, k * LN:(k + 1) * LN])
        if tail:
            tl = jnp.where(lane < tail,
                           x_ref[:, nch * LN:(nch + 1) * LN], -jnp.inf)
            bm = jnp.maximum(bm, tl)
        new_m = jnp.maximum(m_s[...], bm)
        s_s[...] = s_s[...] * jnp.exp(m_s[...] - new_m)
        m_s[...] = new_m

        se = s_s[...]
        rs = rs_s[...]
        xy = xy_s[...]
        yb = yv - j * BC
        for k in range(nch + (1 if tail else 0)):
            raw = x_ref[:, k * LN:(k + 1) * LN]
            if k == nch:
                ch_e = jnp.where(lane < tail, raw, -jnp.inf)
                ch_z = jnp.where(lane < tail, raw, 0.0)
            else:
                ch_e = ch_z = raw
            se = se + jnp.exp(ch_e - new_m)
            rs = rs + ch_z
            xy = xy + jnp.where(lane == yb - k * LN, ch_z, 0.0)
        s_s[...] = se
        rs_s[...] = rs
        xy_s[...] = xy

    @pl.when(j < ncb - 1)
    def _full():
        block(BC // LN, 0)

    @pl.when(j == ncb - 1)
    def _partial():
        rem = c - (ncb - 1) * BC
        block(rem // LN, rem % LN)

        eps = SMOOTH / (c - 2)
        k_const = SMOOTH * jnp.log(jnp.float32(eps)) + CONF * jnp.log(
            jnp.float32(CONF))
        mm = m_s[...]
        big_m = jnp.max(mm, axis=1, keepdims=True)
        s = jnp.sum(s_s[...] * jnp.exp(mm - big_m), axis=1, keepdims=True)
        lse = big_m + jnp.log(s)
        xyv = jnp.sum(xy_s[...], axis=1, keepdims=True)
        rowsum = jnp.sum(rs_s[...], axis=1, keepdims=True)
        rest = rowsum - x0_s[...] - xyv - (c - 2) * lse
        row = k_const - eps * rest - CONF * (xyv - lse)
        row = jnp.where(yv != PAD, row, 0.0)
        out_ref[...] = jnp.sum(row, keepdims=True)[None]


@jax.jit
def kernel(x, y):
    b, c = x.shape
    ncb = pl.cdiv(c, BC)
    nrb = b // BR
    y2 = y.astype(jnp.int32).reshape(b, 1)
    parts = pl.pallas_call(
        functools.partial(_loss_kernel, c=c, ncb=ncb),
        grid=(nrb, ncb),
        in_specs=[
            pl.BlockSpec((BR, BC), lambda i, j: (i, j)),
            pl.BlockSpec((BR, 1), lambda i, j: (i, 0)),
        ],
        out_specs=pl.BlockSpec((1, 1, 1), lambda i, j: (i, 0, 0)),
        out_shape=jax.ShapeDtypeStruct((nrb, 1, 1), jnp.float32),
        scratch_shapes=[
            pltpu.VMEM((BR, LN), jnp.float32),
            pltpu.VMEM((BR, LN), jnp.float32),
            pltpu.VMEM((BR, LN), jnp.float32),
            pltpu.VMEM((BR, LN), jnp.float32),
            pltpu.VMEM((BR, 1), jnp.float32),
        ],
        compiler_params=pltpu.CompilerParams(
            dimension_semantics=("parallel", "arbitrary")),
    )(x, y2)
    return jnp.sum(parts)


# speedup vs baseline: 1.1669x; 1.0087x over previous
import functools

import jax
import jax.numpy as jnp
from jax.experimental import pallas as pl
from jax.experimental.pallas import tpu as pltpu

SMOOTH = 0.1
PAD = 0
CONF = 1.0 - SMOOTH

BR = 256
BC = 10240
LN = 128


def _loss_kernel(x_ref, y_ref, out_ref, m_s, s_s, rs_s, xy_s, x0_s, *, c, ncb):
    j = pl.program_id(1)

    @pl.when(j == 0)
    def _init():
        s_s[...] = jnp.zeros_like(s_s)
        rs_s[...] = jnp.zeros_like(rs_s)
        xy_s[...] = jnp.zeros_like(xy_s)
        m_s[...] = jnp.full_like(m_s, -jnp.inf)
        x0_s[...] = x_ref[:, 0:1]

    yv = y_ref[...]
    lane = jax.lax.broadcasted_iota(jnp.int32, (BR, LN), 1)

    def block(nch, tail):
        bm = x_ref[:, 0:LN]
        for k in range(1, nch):
            bm = jnp.maximum(bm, x_ref[:, k * LN:(k + 1) * LN])
        if tail:
            tl = jnp.where(lane < tail,
                           x_ref[:, nch * LN:(nch + 1) * LN], -jnp.inf)
            bm = jnp.maximum(bm, tl)
        new_m = jnp.maximum(m_s[...], bm)
        s_s[...] = s_s[...] * jnp.exp(m_s[...] - new_m)
        m_s[...] = new_m

        se = s_s[...]
        rs = rs_s[...]
        xy = xy_s[...]
        yb = yv - j * BC
        for k in range(nch + (1 if tail else 0)):
            raw = x_ref[:, k * LN:(k + 1) * LN]
            if k == nch:
                ch_e = jnp.where(lane < tail, raw, -jnp.inf)
                ch_z = jnp.where(lane < tail, raw, 0.0)
            else:
                ch_e = ch_z = raw
            se = se + jnp.exp(ch_e - new_m)
            rs = rs + ch_z
            xy = xy + jnp.where(lane == yb - k * LN, ch_z, 0.0)
        s_s[...] = se
        rs_s[...] = rs
        xy_s[...] = xy

    @pl.when(j < ncb - 1)
    def _full():
        block(BC // LN, 0)

    @pl.when(j == ncb - 1)
    def _partial():
        rem = c - (ncb - 1) * BC
        block(rem // LN, rem % LN)

        eps = SMOOTH / (c - 2)
        k_const = SMOOTH * jnp.log(jnp.float32(eps)) + CONF * jnp.log(
            jnp.float32(CONF))
        mm = m_s[...]
        big_m = jnp.max(mm, axis=1, keepdims=True)
        s = jnp.sum(s_s[...] * jnp.exp(mm - big_m), axis=1, keepdims=True)
        lse = big_m + jnp.log(s)
        xyv = jnp.sum(xy_s[...], axis=1, keepdims=True)
        rowsum = jnp.sum(rs_s[...], axis=1, keepdims=True)
        rest = rowsum - x0_s[...] - xyv - (c - 2) * lse
        row = k_const - eps * rest - CONF * (xyv - lse)
        row = jnp.where(yv != PAD, row, 0.0)
        out_ref[...] = jnp.sum(row, keepdims=True)[None]


@jax.jit
def kernel(x, y):
    b, c = x.shape
    ncb = pl.cdiv(c, BC)
    nrb = b // BR
    y2 = y.astype(jnp.int32).reshape(b, 1)
    parts = pl.pallas_call(
        functools.partial(_loss_kernel, c=c, ncb=ncb),
        grid=(nrb, ncb),
        in_specs=[
            pl.BlockSpec((BR, BC), lambda i, j: (i, j)),
            pl.BlockSpec((BR, 1), lambda i, j: (i, 0)),
        ],
        out_specs=pl.BlockSpec((1, 1, 1), lambda i, j: (i, 0, 0)),
        out_shape=jax.ShapeDtypeStruct((nrb, 1, 1), jnp.float32),
        scratch_shapes=[
            pltpu.VMEM((BR, LN), jnp.float32),
            pltpu.VMEM((BR, LN), jnp.float32),
            pltpu.VMEM((BR, LN), jnp.float32),
            pltpu.VMEM((BR, LN), jnp.float32),
            pltpu.VMEM((BR, 1), jnp.float32),
        ],
        compiler_params=pltpu.CompilerParams(
            dimension_semantics=("parallel", "arbitrary")),
    )(x, y2)
    return jnp.sum(parts)
